# baseline (speedup 1.0000x reference)
import functools

import jax
import jax.numpy as jnp
from jax import lax
from jax.experimental import pallas as pl
from jax.experimental.pallas import tpu as pltpu
from jax.experimental.pallas import tpu_sc as plsc

N_NODES = 10000
N_EDGES = 320000
IN_FEATS = 128
H_FEATS = 256
NUM_CLASSES = 64

NC = 2
NS = 16
NW = NC * NS
CHUNK = 40
NB = 4
E_PAD = 327680
TOTAL_CHUNKS = E_PAD // CHUNK
ACC_ROWS = 10112
ZROWS_PER_TILE = ACC_ROWS // NS
IDX_STAGE = 32
CORE0_CHUNKS = 512
CORE1_CHUNKS = 0
assert NS * (CORE0_CHUNKS + CORE1_CHUNKS) == TOTAL_CHUNKS


def _sc_agg_body(F, x_hbm, src_hbm, dst_hbm, z_big, z_deg, ms0, dg0,
                 acc, dacc, src_v, dst_v, rb0, rb1, rb2, rb3, ones_v,
                 gs0, gs1, gs2, gs3, dsem):
    c = lax.axis_index("c")
    s = lax.axis_index("s")
    rows = [rb0, rb1, rb2, rb3]
    gsem = [gs0, gs1, gs2, gs3]

    one16 = jnp.ones((16,), jnp.float32)
    for i in range(CHUNK):
        ones_v[i, pl.ds(0, 16)] = one16

    r0 = s * ZROWS_PER_TILE

    @pl.when(c == 0)
    def _():
        pltpu.sync_copy(z_big, rb0)
        for k in range(ZROWS_PER_TILE // CHUNK):
            pltpu.sync_copy(rb0, acc.at[pl.ds(r0 + k * CHUNK, CHUNK)])
        rem = ZROWS_PER_TILE % CHUNK
        pltpu.sync_copy(rb0.at[pl.ds(0, rem)],
                        acc.at[pl.ds(r0 + ZROWS_PER_TILE - rem, rem)])
        pltpu.sync_copy(z_deg, dacc.at[pl.ds(r0, ZROWS_PER_TILE)])

    plsc.subcore_barrier()

    tile_base = s * CORE0_CHUNKS
    n_stages = jnp.where(c == 0, CORE0_CHUNKS // IDX_STAGE, 0)

    def chunk_loop(k, carry):
        for b in range(NB):
            j = NB * k + b
            bn = (b + NB - 1) % NB

            @pl.when(j + NB - 1 < IDX_STAGE)
            def _(j=j, bn=bn):
                pltpu.async_copy(x_hbm.at[src_v.at[j + NB - 1]],
                                 rows[bn], gsem[bn])

            pltpu.make_async_copy(x_hbm.at[src_v.at[j]], rows[b],
                                  gsem[b]).wait()
            d = pltpu.async_copy(ones_v, dacc.at[dst_v.at[j]], dsem, add=True)
            pltpu.sync_copy(rows[b], acc.at[dst_v.at[j]], add=True)
            d.wait()
        return carry

    def stage_loop(h, carry):
        row0 = tile_base + h * IDX_STAGE
        pltpu.sync_copy(src_hbm.at[pl.ds(row0, IDX_STAGE)], src_v)
        pltpu.sync_copy(dst_hbm.at[pl.ds(row0, IDX_STAGE)], dst_v)
        for b in range(NB - 1):
            pltpu.async_copy(x_hbm.at[src_v.at[b]], rows[b], gsem[b])
        lax.fori_loop(0, IDX_STAGE // NB, chunk_loop, 0)
        return carry

    lax.fori_loop(0, n_stages, stage_loop, 0)

    plsc.subcore_barrier()

    @pl.when(c == 0)
    def _():
        pltpu.sync_copy(acc.at[pl.ds(r0, ZROWS_PER_TILE)],
                        ms0.at[pl.ds(r0, ZROWS_PER_TILE)])
        pltpu.sync_copy(dacc.at[pl.ds(r0, ZROWS_PER_TILE)],
                        dg0.at[pl.ds(r0, ZROWS_PER_TILE)])


def _make_sc_agg(F):
    mesh = plsc.VectorSubcoreMesh(core_axis_name="c", subcore_axis_name="s",
                                  num_cores=NC, num_subcores=NS)
    return pl.kernel(
        functools.partial(_sc_agg_body, F),
        out_type=[
            jax.ShapeDtypeStruct((ACC_ROWS, F), jnp.float32),
            jax.ShapeDtypeStruct((ACC_ROWS, 16), jnp.float32),
        ],
        mesh=mesh,
        scratch_types=[
            pltpu.VMEM_SHARED((ACC_ROWS, F), jnp.float32),
            pltpu.VMEM_SHARED((ACC_ROWS, 16), jnp.float32),
            pltpu.VMEM((IDX_STAGE, CHUNK), jnp.int32),
            pltpu.VMEM((IDX_STAGE, CHUNK), jnp.int32),
            pltpu.VMEM((CHUNK, F), jnp.float32),
            pltpu.VMEM((CHUNK, F), jnp.float32),
            pltpu.VMEM((CHUNK, F), jnp.float32),
            pltpu.VMEM((CHUNK, F), jnp.float32),
            pltpu.VMEM((CHUNK, 16), jnp.float32),
            pltpu.SemaphoreType.DMA,
            pltpu.SemaphoreType.DMA,
            pltpu.SemaphoreType.DMA,
            pltpu.SemaphoreType.DMA,
            pltpu.SemaphoreType.DMA,
        ],
        compiler_params=pltpu.CompilerParams(use_tc_tiling_on_sc=False),
    )


_sc_agg_128 = _make_sc_agg(IN_FEATS)
_sc_agg_64 = _make_sc_agg(NUM_CLASSES)


def _tc1_body(x_ref, ms_ref, dg_ref,
              w1s_ref, w1n_ref, b1_ref, w2s_ref, w2n_ref, b2_ref,
              p2_ref, s2_ref):
    deg = jnp.maximum(dg_ref[:, 0:1], 1.0)
    h_n = ms_ref[...] / deg
    h = (jnp.dot(x_ref[...], w1s_ref[...], preferred_element_type=jnp.float32)
         + jnp.dot(h_n, w1n_ref[...], preferred_element_type=jnp.float32)
         + b1_ref[...])
    h = jnp.maximum(h, 0.0)
    p2_ref[...] = jnp.dot(h, w2n_ref[...], preferred_element_type=jnp.float32)
    s2_ref[...] = (jnp.dot(h, w2s_ref[...], preferred_element_type=jnp.float32)
                   + b2_ref[...])


def _tc2_body(s2_ref, ms_ref, dg_ref, out_ref):
    deg = jnp.maximum(dg_ref[:, 0:1], 1.0)
    out_ref[...] = s2_ref[...] + ms_ref[...] / deg


_TC_ROWS = 1000


def _tc1(x, ms, dg, w1s, w1n, b1, w2s, w2n, b2):
    grid = (N_NODES // _TC_ROWS,)
    row_block = lambda f: pl.BlockSpec((_TC_ROWS, f), lambda i: (i, 0))
    full = lambda a, b: pl.BlockSpec((a, b), lambda i: (0, 0))
    return pl.pallas_call(
        _tc1_body,
        grid=grid,
        in_specs=[
            row_block(IN_FEATS), row_block(IN_FEATS),
            row_block(16),
            full(IN_FEATS, H_FEATS), full(IN_FEATS, H_FEATS), full(1, H_FEATS),
            full(H_FEATS, NUM_CLASSES), full(H_FEATS, NUM_CLASSES),
            full(1, NUM_CLASSES),
        ],
        out_specs=[row_block(NUM_CLASSES), row_block(NUM_CLASSES)],
        out_shape=[
            jax.ShapeDtypeStruct((N_NODES, NUM_CLASSES), jnp.float32),
            jax.ShapeDtypeStruct((N_NODES, NUM_CLASSES), jnp.float32),
        ],
    )(x, ms, dg, w1s, w1n, b1, w2s, w2n, b2)


def _tc2(s2, ms, dg):
    grid = (N_NODES // _TC_ROWS,)
    row_block = lambda f: pl.BlockSpec((_TC_ROWS, f), lambda i: (i, 0))
    return pl.pallas_call(
        _tc2_body,
        grid=grid,
        in_specs=[
            row_block(NUM_CLASSES), row_block(NUM_CLASSES), row_block(16),
        ],
        out_specs=row_block(NUM_CLASSES),
        out_shape=jax.ShapeDtypeStruct((N_NODES, NUM_CLASSES), jnp.float32),
    )(s2, ms, dg)


def _pack_edges(edge_index):
    src = edge_index[0].astype(jnp.int32)
    dst = edge_index[1].astype(jnp.int32)
    pad = E_PAD - N_EDGES
    src = jnp.concatenate([src, jnp.zeros((pad,), jnp.int32)])
    junk = N_NODES + (jnp.arange(pad, dtype=jnp.int32) % (ACC_ROWS - N_NODES))
    dst = jnp.concatenate([dst, junk])
    return src.reshape(TOTAL_CHUNKS, CHUNK), dst.reshape(TOTAL_CHUNKS, CHUNK)


def kernel(x, edge_index1, edge_index2, W1, b1, W2, b2):
    sp1, dp1 = _pack_edges(edge_index1)
    sp2, dp2 = _pack_edges(edge_index2)

    w1s = W1[:, :IN_FEATS].T
    w1n = W1[:, IN_FEATS:].T
    w2s = W2[:, :H_FEATS].T
    w2n = W2[:, H_FEATS:].T
    b1r = b1.reshape(1, H_FEATS)
    b2r = b2.reshape(1, NUM_CLASSES)

    z128 = jnp.zeros((CHUNK, IN_FEATS), jnp.float32)
    z64 = jnp.zeros((CHUNK, NUM_CLASSES), jnp.float32)
    z16 = jnp.zeros((ZROWS_PER_TILE, 16), jnp.float32)

    ms1_, dg1_ = _sc_agg_128(x, sp1, dp1, z128, z16)
    p2, s2 = _tc1(x, ms1_, dg1_, w1s, w1n, b1r, w2s, w2n, b2r)
    ms2_, dg2_ = _sc_agg_64(p2, sp2, dp2, z64, z16)
    return _tc2(s2, ms2_, dg2_)


# speedup vs baseline: 3.1357x; 3.1357x over previous
import functools

import jax
import jax.numpy as jnp
from jax import lax
from jax.experimental import pallas as pl
from jax.experimental.pallas import tpu as pltpu
from jax.experimental.pallas import tpu_sc as plsc

N_NODES = 10000
N_EDGES = 320000
IN_FEATS = 128
H_FEATS = 256
NUM_CLASSES = 64

NC = 2
NS = 16
CHUNK = 40
NB = 4
E_PAD = 327680
TOTAL_CHUNKS = E_PAD // CHUNK
ACC_ROWS = 10112
ZROWS_PER_TILE = ACC_ROWS // NS
IDX_STAGE = 32
CORE_CHUNKS = TOTAL_CHUNKS // NC // NS


def _sc_agg_body(F, x_hbm, src_hbm, dst_hbm, z_big, z_deg, ms0, ms1, dg0, dg1,
                 acc, dacc, src_v, dst_v, rb0, rb1, rb2, rb3, ones_v,
                 gs0, gs1, gs2, gs3, dsem):
    c = lax.axis_index("c")
    s = lax.axis_index("s")
    rows = [rb0, rb1, rb2, rb3]
    gsem = [gs0, gs1, gs2, gs3]

    one16 = jnp.ones((16,), jnp.float32)
    for i in range(CHUNK):
        ones_v[i, pl.ds(0, 16)] = one16

    r0 = s * ZROWS_PER_TILE
    pltpu.sync_copy(z_big, rb0)
    for k in range(ZROWS_PER_TILE // CHUNK):
        pltpu.sync_copy(rb0, acc.at[pl.ds(r0 + k * CHUNK, CHUNK)])
    rem = ZROWS_PER_TILE % CHUNK
    pltpu.sync_copy(rb0.at[pl.ds(0, rem)],
                    acc.at[pl.ds(r0 + ZROWS_PER_TILE - rem, rem)])
    pltpu.sync_copy(z_deg, dacc.at[pl.ds(r0, ZROWS_PER_TILE)])

    plsc.subcore_barrier()

    tile_base = (c * NS + s) * CORE_CHUNKS

    def chunk_loop(k, carry):
        for b in range(NB):
            j = NB * k + b
            bn = (b + NB - 1) % NB

            @pl.when(j + NB - 1 < IDX_STAGE)
            def _(j=j, bn=bn):
                pltpu.async_copy(x_hbm.at[src_v.at[j + NB - 1]],
                                 rows[bn], gsem[bn])

            pltpu.make_async_copy(x_hbm.at[src_v.at[j]], rows[b],
                                  gsem[b]).wait()
            d = pltpu.async_copy(ones_v, dacc.at[dst_v.at[j]], dsem, add=True)
            pltpu.sync_copy(rows[b], acc.at[dst_v.at[j]], add=True)
            d.wait()
        return carry

    def stage_loop(h, carry):
        row0 = tile_base + h * IDX_STAGE
        pltpu.sync_copy(src_hbm.at[pl.ds(row0, IDX_STAGE)], src_v)
        pltpu.sync_copy(dst_hbm.at[pl.ds(row0, IDX_STAGE)], dst_v)
        for b in range(NB - 1):
            pltpu.async_copy(x_hbm.at[src_v.at[b]], rows[b], gsem[b])
        lax.fori_loop(0, IDX_STAGE // NB, chunk_loop, 0)
        return carry

    lax.fori_loop(0, CORE_CHUNKS // IDX_STAGE, stage_loop, 0)

    plsc.subcore_barrier()

    @pl.when(c == 0)
    def _():
        pltpu.sync_copy(acc.at[pl.ds(r0, ZROWS_PER_TILE)],
                        ms0.at[pl.ds(r0, ZROWS_PER_TILE)])
        pltpu.sync_copy(dacc.at[pl.ds(r0, ZROWS_PER_TILE)],
                        dg0.at[pl.ds(r0, ZROWS_PER_TILE)])

    @pl.when(c == 1)
    def _():
        pltpu.sync_copy(acc.at[pl.ds(r0, ZROWS_PER_TILE)],
                        ms1.at[pl.ds(r0, ZROWS_PER_TILE)])
        pltpu.sync_copy(dacc.at[pl.ds(r0, ZROWS_PER_TILE)],
                        dg1.at[pl.ds(r0, ZROWS_PER_TILE)])


def _make_sc_agg(F):
    mesh = plsc.VectorSubcoreMesh(core_axis_name="c", subcore_axis_name="s",
                                  num_cores=NC, num_subcores=NS)
    return pl.kernel(
        functools.partial(_sc_agg_body, F),
        out_type=[
            jax.ShapeDtypeStruct((ACC_ROWS, F), jnp.float32),
            jax.ShapeDtypeStruct((ACC_ROWS, F), jnp.float32),
            jax.ShapeDtypeStruct((ACC_ROWS, 16), jnp.float32),
            jax.ShapeDtypeStruct((ACC_ROWS, 16), jnp.float32),
        ],
        mesh=mesh,
        scratch_types=[
            pltpu.VMEM_SHARED((ACC_ROWS, F), jnp.float32),
            pltpu.VMEM_SHARED((ACC_ROWS, 16), jnp.float32),
            pltpu.VMEM((IDX_STAGE, CHUNK), jnp.int32),
            pltpu.VMEM((IDX_STAGE, CHUNK), jnp.int32),
            pltpu.VMEM((CHUNK, F), jnp.float32),
            pltpu.VMEM((CHUNK, F), jnp.float32),
            pltpu.VMEM((CHUNK, F), jnp.float32),
            pltpu.VMEM((CHUNK, F), jnp.float32),
            pltpu.VMEM((CHUNK, 16), jnp.float32),
            pltpu.SemaphoreType.DMA,
            pltpu.SemaphoreType.DMA,
            pltpu.SemaphoreType.DMA,
            pltpu.SemaphoreType.DMA,
            pltpu.SemaphoreType.DMA,
        ],
        compiler_params=pltpu.CompilerParams(use_tc_tiling_on_sc=False),
    )


_sc_agg_128 = _make_sc_agg(IN_FEATS)
_sc_agg_64 = _make_sc_agg(NUM_CLASSES)


def _tc1_body(x_ref, ms0_ref, ms1_ref, dg0_ref, dg1_ref,
              w1s_ref, w1n_ref, b1_ref, w2s_ref, w2n_ref, b2_ref,
              p2_ref, s2_ref):
    deg = jnp.maximum(dg0_ref[:, 0:1] + dg1_ref[:, 0:1], 1.0)
    h_n = (ms0_ref[...] + ms1_ref[...]) / deg
    h = (jnp.dot(x_ref[...], w1s_ref[...], preferred_element_type=jnp.float32)
         + jnp.dot(h_n, w1n_ref[...], preferred_element_type=jnp.float32)
         + b1_ref[...])
    h = jnp.maximum(h, 0.0)
    p2_ref[...] = jnp.dot(h, w2n_ref[...], preferred_element_type=jnp.float32)
    s2_ref[...] = (jnp.dot(h, w2s_ref[...], preferred_element_type=jnp.float32)
                   + b2_ref[...])


def _tc2_body(s2_ref, ms0_ref, ms1_ref, dg0_ref, dg1_ref, out_ref):
    deg = jnp.maximum(dg0_ref[:, 0:1] + dg1_ref[:, 0:1], 1.0)
    out_ref[...] = s2_ref[...] + (ms0_ref[...] + ms1_ref[...]) / deg


_TC_ROWS = 1000


def _tc1(x, ms0, ms1, dg0, dg1, w1s, w1n, b1, w2s, w2n, b2):
    grid = (N_NODES // _TC_ROWS,)
    row_block = lambda f: pl.BlockSpec((_TC_ROWS, f), lambda i: (i, 0))
    full = lambda a, b: pl.BlockSpec((a, b), lambda i: (0, 0))
    return pl.pallas_call(
        _tc1_body,
        grid=grid,
        in_specs=[
            row_block(IN_FEATS), row_block(IN_FEATS), row_block(IN_FEATS),
            row_block(16), row_block(16),
            full(IN_FEATS, H_FEATS), full(IN_FEATS, H_FEATS), full(1, H_FEATS),
            full(H_FEATS, NUM_CLASSES), full(H_FEATS, NUM_CLASSES),
            full(1, NUM_CLASSES),
        ],
        out_specs=[row_block(NUM_CLASSES), row_block(NUM_CLASSES)],
        out_shape=[
            jax.ShapeDtypeStruct((N_NODES, NUM_CLASSES), jnp.float32),
            jax.ShapeDtypeStruct((N_NODES, NUM_CLASSES), jnp.float32),
        ],
    )(x, ms0, ms1, dg0, dg1, w1s, w1n, b1, w2s, w2n, b2)


def _tc2(s2, ms0, ms1, dg0, dg1):
    grid = (N_NODES // _TC_ROWS,)
    row_block = lambda f: pl.BlockSpec((_TC_ROWS, f), lambda i: (i, 0))
    return pl.pallas_call(
        _tc2_body,
        grid=grid,
        in_specs=[
            row_block(NUM_CLASSES), row_block(NUM_CLASSES),
            row_block(NUM_CLASSES), row_block(16), row_block(16),
        ],
        out_specs=row_block(NUM_CLASSES),
        out_shape=jax.ShapeDtypeStruct((N_NODES, NUM_CLASSES), jnp.float32),
    )(s2, ms0, ms1, dg0, dg1)


def _pack_edges(edge_index):
    src = edge_index[0].astype(jnp.int32)
    dst = edge_index[1].astype(jnp.int32)
    pad = E_PAD - N_EDGES
    ar = jnp.arange(pad, dtype=jnp.int32)
    src = jnp.concatenate([src, (ar * 97) % N_NODES])
    dst = jnp.concatenate([dst, N_NODES + ar % (ACC_ROWS - N_NODES)])
    return src.reshape(TOTAL_CHUNKS, CHUNK), dst.reshape(TOTAL_CHUNKS, CHUNK)


def kernel(x, edge_index1, edge_index2, W1, b1, W2, b2):
    sp1, dp1 = _pack_edges(edge_index1)
    sp2, dp2 = _pack_edges(edge_index2)

    w1s = W1[:, :IN_FEATS].T
    w1n = W1[:, IN_FEATS:].T
    w2s = W2[:, :H_FEATS].T
    w2n = W2[:, H_FEATS:].T
    b1r = b1.reshape(1, H_FEATS)
    b2r = b2.reshape(1, NUM_CLASSES)

    z128 = jnp.zeros((CHUNK, IN_FEATS), jnp.float32)
    z64 = jnp.zeros((CHUNK, NUM_CLASSES), jnp.float32)
    z16 = jnp.zeros((ZROWS_PER_TILE, 16), jnp.float32)

    ms10, ms11, dg10, dg11 = _sc_agg_128(x, sp1, dp1, z128, z16)
    p2, s2 = _tc1(x, ms10, ms11, dg10, dg11, w1s, w1n, b1r, w2s, w2n, b2r)
    ms20, ms21, dg20, dg21 = _sc_agg_64(p2, sp2, dp2, z64, z16)
    return _tc2(s2, ms20, ms21, dg20, dg21)


# speedup vs baseline: 3.6804x; 1.1737x over previous
import functools

import jax
import jax.numpy as jnp
from jax import lax
from jax.experimental import pallas as pl
from jax.experimental.pallas import tpu as pltpu
from jax.experimental.pallas import tpu_sc as plsc

N_NODES = 10000
N_EDGES = 320000
IN_FEATS = 128
H_FEATS = 256
NUM_CLASSES = 64

NC = 2
NS = 16
CHUNK = 40
NB = 5
TOTAL_CHUNKS = N_EDGES // CHUNK
ACC_ROWS = 10112
ZROWS_PER_TILE = ACC_ROWS // NS
IDX_STAGE = 50
CORE_CHUNKS = TOTAL_CHUNKS // NC // NS


def _sc_agg_body(F, x_hbm, e_hbm, z_big, z_deg, ms0, ms1, dg0, dg1,
                 acc, dacc, src_v, dst_v, rb0, rb1, rb2, rb3, rb4, ones_v,
                 gs0, gs1, gs2, gs3, gs4, dsem):
    c = lax.axis_index("c")
    s = lax.axis_index("s")
    rows = [rb0, rb1, rb2, rb3, rb4]
    gsem = [gs0, gs1, gs2, gs3, gs4]

    one16 = jnp.ones((16,), jnp.float32)
    for i in range(CHUNK):
        ones_v[i, pl.ds(0, 16)] = one16

    r0 = s * ZROWS_PER_TILE
    pltpu.sync_copy(z_big, rb0)
    for k in range(ZROWS_PER_TILE // CHUNK):
        pltpu.sync_copy(rb0, acc.at[pl.ds(r0 + k * CHUNK, CHUNK)])
    rem = ZROWS_PER_TILE % CHUNK
    pltpu.sync_copy(rb0.at[pl.ds(0, rem)],
                    acc.at[pl.ds(r0 + ZROWS_PER_TILE - rem, rem)])
    pltpu.sync_copy(z_deg, dacc.at[pl.ds(r0, ZROWS_PER_TILE)])

    plsc.subcore_barrier()

    tile_base = (c * NS + s) * CORE_CHUNKS

    def chunk_loop(k, carry):
        for b in range(NB):
            j = NB * k + b
            bn = (b + NB - 1) % NB

            @pl.when(j + NB - 1 < IDX_STAGE)
            def _(j=j, bn=bn):
                pltpu.async_copy(x_hbm.at[src_v.at[j + NB - 1]],
                                 rows[bn], gsem[bn])

            pltpu.make_async_copy(x_hbm.at[src_v.at[j]], rows[b],
                                  gsem[b]).wait()
            d = pltpu.async_copy(ones_v, dacc.at[dst_v.at[j]], dsem, add=True)
            pltpu.sync_copy(rows[b], acc.at[dst_v.at[j]], add=True)
            d.wait()
        return carry

    def stage_loop(h, carry):
        row0 = tile_base + h * IDX_STAGE
        pltpu.sync_copy(e_hbm.at[0, pl.ds(row0, IDX_STAGE)], src_v)
        pltpu.sync_copy(e_hbm.at[1, pl.ds(row0, IDX_STAGE)], dst_v)
        for b in range(NB - 1):
            pltpu.async_copy(x_hbm.at[src_v.at[b]], rows[b], gsem[b])
        lax.fori_loop(0, IDX_STAGE // NB, chunk_loop, 0)
        return carry

    lax.fori_loop(0, CORE_CHUNKS // IDX_STAGE, stage_loop, 0)

    plsc.subcore_barrier()

    @pl.when(c == 0)
    def _():
        pltpu.sync_copy(acc.at[pl.ds(r0, ZROWS_PER_TILE)],
                        ms0.at[pl.ds(r0, ZROWS_PER_TILE)])
        pltpu.sync_copy(dacc.at[pl.ds(r0, ZROWS_PER_TILE)],
                        dg0.at[pl.ds(r0, ZROWS_PER_TILE)])

    @pl.when(c == 1)
    def _():
        pltpu.sync_copy(acc.at[pl.ds(r0, ZROWS_PER_TILE)],
                        ms1.at[pl.ds(r0, ZROWS_PER_TILE)])
        pltpu.sync_copy(dacc.at[pl.ds(r0, ZROWS_PER_TILE)],
                        dg1.at[pl.ds(r0, ZROWS_PER_TILE)])


def _make_sc_agg(F):
    mesh = plsc.VectorSubcoreMesh(core_axis_name="c", subcore_axis_name="s",
                                  num_cores=NC, num_subcores=NS)
    return pl.kernel(
        functools.partial(_sc_agg_body, F),
        out_type=[
            jax.ShapeDtypeStruct((ACC_ROWS, F), jnp.float32),
            jax.ShapeDtypeStruct((ACC_ROWS, F), jnp.float32),
            jax.ShapeDtypeStruct((ACC_ROWS, 16), jnp.float32),
            jax.ShapeDtypeStruct((ACC_ROWS, 16), jnp.float32),
        ],
        mesh=mesh,
        scratch_types=[
            pltpu.VMEM_SHARED((ACC_ROWS, F), jnp.float32),
            pltpu.VMEM_SHARED((ACC_ROWS, 16), jnp.float32),
            pltpu.VMEM((IDX_STAGE, CHUNK), jnp.int32),
            pltpu.VMEM((IDX_STAGE, CHUNK), jnp.int32),
            pltpu.VMEM((CHUNK, F), jnp.float32),
            pltpu.VMEM((CHUNK, F), jnp.float32),
            pltpu.VMEM((CHUNK, F), jnp.float32),
            pltpu.VMEM((CHUNK, F), jnp.float32),
            pltpu.VMEM((CHUNK, F), jnp.float32),
            pltpu.VMEM((CHUNK, 16), jnp.float32),
            pltpu.SemaphoreType.DMA,
            pltpu.SemaphoreType.DMA,
            pltpu.SemaphoreType.DMA,
            pltpu.SemaphoreType.DMA,
            pltpu.SemaphoreType.DMA,
            pltpu.SemaphoreType.DMA,
        ],
        compiler_params=pltpu.CompilerParams(use_tc_tiling_on_sc=False),
    )


_sc_agg_128 = _make_sc_agg(IN_FEATS)
_sc_agg_64 = _make_sc_agg(NUM_CLASSES)


def _tc1_body(x_ref, ms0_ref, ms1_ref, dg0_ref, dg1_ref,
              w1s_ref, w1n_ref, b1_ref, w2s_ref, w2n_ref, b2_ref,
              p2_ref, s2_ref):
    deg = jnp.maximum(dg0_ref[:, 0:1] + dg1_ref[:, 0:1], 1.0)
    h_n = (ms0_ref[...] + ms1_ref[...]) / deg
    h = (jnp.dot(x_ref[...], w1s_ref[...], preferred_element_type=jnp.float32)
         + jnp.dot(h_n, w1n_ref[...], preferred_element_type=jnp.float32)
         + b1_ref[...])
    h = jnp.maximum(h, 0.0)
    p2_ref[...] = jnp.dot(h, w2n_ref[...], preferred_element_type=jnp.float32)
    s2_ref[...] = (jnp.dot(h, w2s_ref[...], preferred_element_type=jnp.float32)
                   + b2_ref[...])


def _tc2_body(s2_ref, ms0_ref, ms1_ref, dg0_ref, dg1_ref, out_ref):
    deg = jnp.maximum(dg0_ref[:, 0:1] + dg1_ref[:, 0:1], 1.0)
    out_ref[...] = s2_ref[...] + (ms0_ref[...] + ms1_ref[...]) / deg


_TC_ROWS = 1000


def _tc1(x, ms0, ms1, dg0, dg1, w1s, w1n, b1, w2s, w2n, b2):
    grid = (N_NODES // _TC_ROWS,)
    row_block = lambda f: pl.BlockSpec((_TC_ROWS, f), lambda i: (i, 0))
    full = lambda a, b: pl.BlockSpec((a, b), lambda i: (0, 0))
    return pl.pallas_call(
        _tc1_body,
        grid=grid,
        in_specs=[
            row_block(IN_FEATS), row_block(IN_FEATS), row_block(IN_FEATS),
            row_block(16), row_block(16),
            full(IN_FEATS, H_FEATS), full(IN_FEATS, H_FEATS), full(1, H_FEATS),
            full(H_FEATS, NUM_CLASSES), full(H_FEATS, NUM_CLASSES),
            full(1, NUM_CLASSES),
        ],
        out_specs=[row_block(NUM_CLASSES), row_block(NUM_CLASSES)],
        out_shape=[
            jax.ShapeDtypeStruct((N_NODES, NUM_CLASSES), jnp.float32),
            jax.ShapeDtypeStruct((N_NODES, NUM_CLASSES), jnp.float32),
        ],
    )(x, ms0, ms1, dg0, dg1, w1s, w1n, b1, w2s, w2n, b2)


def _tc2(s2, ms0, ms1, dg0, dg1):
    grid = (N_NODES // _TC_ROWS,)
    row_block = lambda f: pl.BlockSpec((_TC_ROWS, f), lambda i: (i, 0))
    return pl.pallas_call(
        _tc2_body,
        grid=grid,
        in_specs=[
            row_block(NUM_CLASSES), row_block(NUM_CLASSES),
            row_block(NUM_CLASSES), row_block(16), row_block(16),
        ],
        out_specs=row_block(NUM_CLASSES),
        out_shape=jax.ShapeDtypeStruct((N_NODES, NUM_CLASSES), jnp.float32),
    )(s2, ms0, ms1, dg0, dg1)


def _pack_edges(edge_index):
    return edge_index.astype(jnp.int32).reshape(2, TOTAL_CHUNKS, CHUNK)


def kernel(x, edge_index1, edge_index2, W1, b1, W2, b2):
    e1 = _pack_edges(edge_index1)
    e2 = _pack_edges(edge_index2)

    w1s = W1[:, :IN_FEATS].T
    w1n = W1[:, IN_FEATS:].T
    w2s = W2[:, :H_FEATS].T
    w2n = W2[:, H_FEATS:].T
    b1r = b1.reshape(1, H_FEATS)
    b2r = b2.reshape(1, NUM_CLASSES)

    z128 = jnp.zeros((CHUNK, IN_FEATS), jnp.float32)
    z64 = jnp.zeros((CHUNK, NUM_CLASSES), jnp.float32)
    z16 = jnp.zeros((ZROWS_PER_TILE, 16), jnp.float32)

    ms10, ms11, dg10, dg11 = _sc_agg_128(x, e1, z128, z16)
    p2, s2 = _tc1(x, ms10, ms11, dg10, dg11, w1s, w1n, b1r, w2s, w2n, b2r)
    ms20, ms21, dg20, dg21 = _sc_agg_64(p2, e2, z64, z16)
    return _tc2(s2, ms20, ms21, dg20, dg21)


# speedup vs baseline: 3.7270x; 1.0126x over previous
import functools

import jax
import jax.numpy as jnp
from jax import lax
from jax.experimental import pallas as pl
from jax.experimental.pallas import tpu as pltpu
from jax.experimental.pallas import tpu_sc as plsc

N_NODES = 10000
N_EDGES = 320000
IN_FEATS = 128
H_FEATS = 256
NUM_CLASSES = 64

NC = 2
NS = 16
NB = 5
CHUNK_1 = 40
CHUNK_2 = 80
ISTG_1 = 50
ISTG_2 = 25
ACC_ROWS = 10112
ZROWS_PER_TILE = ACC_ROWS // NS


def _sc_agg_body(F, CHUNK, IDX_STAGE, x_hbm, e_hbm, z_big, z_deg, ms0, ms1, dg0, dg1,
                 acc, dacc, src_v, dst_v, rb0, rb1, rb2, rb3, rb4, ones_v,
                 gs0, gs1, gs2, gs3, gs4, dsem):
    c = lax.axis_index("c")
    s = lax.axis_index("s")
    rows = [rb0, rb1, rb2, rb3, rb4]
    gsem = [gs0, gs1, gs2, gs3, gs4]

    one16 = jnp.ones((16,), jnp.float32)
    for i in range(CHUNK):
        ones_v[i, pl.ds(0, 16)] = one16

    r0 = s * ZROWS_PER_TILE
    pltpu.sync_copy(z_big, rb0)
    for k in range(ZROWS_PER_TILE // CHUNK):
        pltpu.sync_copy(rb0, acc.at[pl.ds(r0 + k * CHUNK, CHUNK)])
    rem = ZROWS_PER_TILE % CHUNK
    pltpu.sync_copy(rb0.at[pl.ds(0, rem)],
                    acc.at[pl.ds(r0 + ZROWS_PER_TILE - rem, rem)])
    pltpu.sync_copy(z_deg, dacc.at[pl.ds(r0, ZROWS_PER_TILE)])

    plsc.subcore_barrier()

    core_chunks = N_EDGES // CHUNK // NC // NS
    tile_base = (c * NS + s) * core_chunks

    def chunk_loop(k, carry):
        for b in range(NB):
            j = NB * k + b
            bn = (b + NB - 1) % NB

            @pl.when(j + NB - 1 < IDX_STAGE)
            def _(j=j, bn=bn):
                pltpu.async_copy(x_hbm.at[src_v.at[j + NB - 1]],
                                 rows[bn], gsem[bn])

            pltpu.make_async_copy(x_hbm.at[src_v.at[j]], rows[b],
                                  gsem[b]).wait()
            d = pltpu.async_copy(ones_v, dacc.at[dst_v.at[j]], dsem, add=True)
            pltpu.sync_copy(rows[b], acc.at[dst_v.at[j]], add=True)
            d.wait()
        return carry

    def stage_loop(h, carry):
        row0 = tile_base + h * IDX_STAGE
        pltpu.sync_copy(e_hbm.at[0, pl.ds(row0, IDX_STAGE)], src_v)
        pltpu.sync_copy(e_hbm.at[1, pl.ds(row0, IDX_STAGE)], dst_v)
        for b in range(NB - 1):
            pltpu.async_copy(x_hbm.at[src_v.at[b]], rows[b], gsem[b])
        lax.fori_loop(0, IDX_STAGE // NB, chunk_loop, 0)
        return carry

    lax.fori_loop(0, core_chunks // IDX_STAGE, stage_loop, 0)

    plsc.subcore_barrier()

    @pl.when(c == 0)
    def _():
        pltpu.sync_copy(acc.at[pl.ds(r0, ZROWS_PER_TILE)],
                        ms0.at[pl.ds(r0, ZROWS_PER_TILE)])
        pltpu.sync_copy(dacc.at[pl.ds(r0, ZROWS_PER_TILE)],
                        dg0.at[pl.ds(r0, ZROWS_PER_TILE)])

    @pl.when(c == 1)
    def _():
        pltpu.sync_copy(acc.at[pl.ds(r0, ZROWS_PER_TILE)],
                        ms1.at[pl.ds(r0, ZROWS_PER_TILE)])
        pltpu.sync_copy(dacc.at[pl.ds(r0, ZROWS_PER_TILE)],
                        dg1.at[pl.ds(r0, ZROWS_PER_TILE)])


def _make_sc_agg(F, CHUNK, IDX_STAGE):
    mesh = plsc.VectorSubcoreMesh(core_axis_name="c", subcore_axis_name="s",
                                  num_cores=NC, num_subcores=NS)
    return pl.kernel(
        functools.partial(_sc_agg_body, F, CHUNK, IDX_STAGE),
        out_type=[
            jax.ShapeDtypeStruct((ACC_ROWS, F), jnp.float32),
            jax.ShapeDtypeStruct((ACC_ROWS, F), jnp.float32),
            jax.ShapeDtypeStruct((ACC_ROWS, 16), jnp.float32),
            jax.ShapeDtypeStruct((ACC_ROWS, 16), jnp.float32),
        ],
        mesh=mesh,
        scratch_types=[
            pltpu.VMEM_SHARED((ACC_ROWS, F), jnp.float32),
            pltpu.VMEM_SHARED((ACC_ROWS, 16), jnp.float32),
            pltpu.VMEM((IDX_STAGE, CHUNK), jnp.int32),
            pltpu.VMEM((IDX_STAGE, CHUNK), jnp.int32),
            pltpu.VMEM((CHUNK, F), jnp.float32),
            pltpu.VMEM((CHUNK, F), jnp.float32),
            pltpu.VMEM((CHUNK, F), jnp.float32),
            pltpu.VMEM((CHUNK, F), jnp.float32),
            pltpu.VMEM((CHUNK, F), jnp.float32),
            pltpu.VMEM((CHUNK, 16), jnp.float32),
            pltpu.SemaphoreType.DMA,
            pltpu.SemaphoreType.DMA,
            pltpu.SemaphoreType.DMA,
            pltpu.SemaphoreType.DMA,
            pltpu.SemaphoreType.DMA,
            pltpu.SemaphoreType.DMA,
        ],
        compiler_params=pltpu.CompilerParams(use_tc_tiling_on_sc=False),
    )


_sc_agg_128 = _make_sc_agg(IN_FEATS, CHUNK_1, ISTG_1)
_sc_agg_64 = _make_sc_agg(NUM_CLASSES, CHUNK_2, ISTG_2)


def _tc1_body(x_ref, ms0_ref, ms1_ref, dg0_ref, dg1_ref,
              w1s_ref, w1n_ref, b1_ref, w2s_ref, w2n_ref, b2_ref,
              p2_ref, s2_ref):
    deg = jnp.maximum(dg0_ref[:, 0:1] + dg1_ref[:, 0:1], 1.0)
    h_n = (ms0_ref[...] + ms1_ref[...]) / deg
    h = (jnp.dot(x_ref[...], w1s_ref[...], preferred_element_type=jnp.float32)
         + jnp.dot(h_n, w1n_ref[...], preferred_element_type=jnp.float32)
         + b1_ref[...])
    h = jnp.maximum(h, 0.0)
    p2_ref[...] = jnp.dot(h, w2n_ref[...], preferred_element_type=jnp.float32)
    s2_ref[...] = (jnp.dot(h, w2s_ref[...], preferred_element_type=jnp.float32)
                   + b2_ref[...])


def _tc2_body(s2_ref, ms0_ref, ms1_ref, dg0_ref, dg1_ref, out_ref):
    deg = jnp.maximum(dg0_ref[:, 0:1] + dg1_ref[:, 0:1], 1.0)
    out_ref[...] = s2_ref[...] + (ms0_ref[...] + ms1_ref[...]) / deg


_TC_ROWS = 1000


def _tc1(x, ms0, ms1, dg0, dg1, w1s, w1n, b1, w2s, w2n, b2):
    grid = (N_NODES // _TC_ROWS,)
    row_block = lambda f: pl.BlockSpec((_TC_ROWS, f), lambda i: (i, 0))
    full = lambda a, b: pl.BlockSpec((a, b), lambda i: (0, 0))
    return pl.pallas_call(
        _tc1_body,
        grid=grid,
        in_specs=[
            row_block(IN_FEATS), row_block(IN_FEATS), row_block(IN_FEATS),
            row_block(16), row_block(16),
            full(IN_FEATS, H_FEATS), full(IN_FEATS, H_FEATS), full(1, H_FEATS),
            full(H_FEATS, NUM_CLASSES), full(H_FEATS, NUM_CLASSES),
            full(1, NUM_CLASSES),
        ],
        out_specs=[row_block(NUM_CLASSES), row_block(NUM_CLASSES)],
        out_shape=[
            jax.ShapeDtypeStruct((N_NODES, NUM_CLASSES), jnp.float32),
            jax.ShapeDtypeStruct((N_NODES, NUM_CLASSES), jnp.float32),
        ],
    )(x, ms0, ms1, dg0, dg1, w1s, w1n, b1, w2s, w2n, b2)


def _tc2(s2, ms0, ms1, dg0, dg1):
    grid = (N_NODES // _TC_ROWS,)
    row_block = lambda f: pl.BlockSpec((_TC_ROWS, f), lambda i: (i, 0))
    return pl.pallas_call(
        _tc2_body,
        grid=grid,
        in_specs=[
            row_block(NUM_CLASSES), row_block(NUM_CLASSES),
            row_block(NUM_CLASSES), row_block(16), row_block(16),
        ],
        out_specs=row_block(NUM_CLASSES),
        out_shape=jax.ShapeDtypeStruct((N_NODES, NUM_CLASSES), jnp.float32),
    )(s2, ms0, ms1, dg0, dg1)


def _pack_edges(edge_index, chunk):
    return edge_index.astype(jnp.int32).reshape(2, N_EDGES // chunk, chunk)


def kernel(x, edge_index1, edge_index2, W1, b1, W2, b2):
    e1 = _pack_edges(edge_index1, CHUNK_1)
    e2 = _pack_edges(edge_index2, CHUNK_2)

    w1s = W1[:, :IN_FEATS].T
    w1n = W1[:, IN_FEATS:].T
    w2s = W2[:, :H_FEATS].T
    w2n = W2[:, H_FEATS:].T
    b1r = b1.reshape(1, H_FEATS)
    b2r = b2.reshape(1, NUM_CLASSES)

    z128 = jnp.zeros((CHUNK_1, IN_FEATS), jnp.float32)
    z64 = jnp.zeros((CHUNK_2, NUM_CLASSES), jnp.float32)
    z16 = jnp.zeros((ZROWS_PER_TILE, 16), jnp.float32)

    ms10, ms11, dg10, dg11 = _sc_agg_128(x, e1, z128, z16)
    p2, s2 = _tc1(x, ms10, ms11, dg10, dg11, w1s, w1n, b1r, w2s, w2n, b2r)
    ms20, ms21, dg20, dg21 = _sc_agg_64(p2, e2, z64, z16)
    return _tc2(s2, ms20, ms21, dg20, dg21)
